# baseline (speedup 1.0000x reference)
import jax
import jax.numpy as jnp
from jax import lax
from jax.experimental import pallas as pl
from jax.experimental.pallas import tpu as pltpu
from jax.experimental.pallas import tpu_sc as plsc

N = 16
HW = 128 * 128
TOT = N * HW
P = 8192
NC, NS, L = 2, 16, 16
NW = NC * NS
PPW = P // NW
CH = 2
CB = PPW // CH
EPS = 1e-6
ZROWS = TOT // NS


def _sc_body(boxes_hbm, gt_hbm, bidx_hbm, midx_hbm, resp_hbm, gsum_hbm,
             bidx_v, midx_v, pidx_v, sidx_v, pc_v, iou_v, gt_vm, gacc_v,
             zbuf, sem):
    cid = lax.axis_index("c")
    sid = lax.axis_index("s")
    wid = sid * NC + cid

    pltpu.sync_copy(gt_hbm, gt_vm)
    pltpu.sync_copy(bidx_hbm.at[pl.ds(wid * CH, CH)], bidx_v)
    pltpu.sync_copy(midx_hbm.at[pl.ds(wid * CH, CH)], midx_v)

    def _z(i, c):
        for k in range(8):
            zbuf[pl.ds(i * 128 + k * L, L)] = jnp.zeros((L,), jnp.float32)
        return c
    lax.fori_loop(0, ZROWS // 128, _z, 0)
    pltpu.sync_copy(zbuf, resp_hbm.at[pl.ds(cid * TOT + sid * ZROWS, ZROWS)])

    for j in range(CH):
        for k in range(CB // L):
            b = bidx_v[j, pl.ds(k * L, L)]
            m = midx_v[j, pl.ds(k * L, L)]
            f = b * HW + m
            sidx_v[j, pl.ds(k * L, L)] = f + cid * TOT
            f4 = f * 4
            for c in range(4):
                pidx_v[j * 4 + c, pl.ds(k * L, L)] = f4 + c

    copies = [
        pltpu.async_copy(boxes_hbm.at[pidx_v.at[j * 4 + c]],
                         pc_v.at[j * 4 + c], sem)
        for j in range(CH) for c in range(4)
    ]
    for cp in copies:
        cp.wait()

    plsc.subcore_barrier()

    gacc = jnp.zeros((L,), jnp.float32)
    for j in range(CH):
        for k in range(CB // L):
            sl = pl.ds(k * L, L)
            bvec = bidx_v[j, sl] * 4
            px1 = pc_v[j * 4 + 0, sl]
            py1 = pc_v[j * 4 + 1, sl]
            px2 = pc_v[j * 4 + 2, sl]
            py2 = pc_v[j * 4 + 3, sl]
            gx1 = plsc.load_gather(gt_vm, [bvec])
            gy1 = plsc.load_gather(gt_vm, [bvec + 1])
            gx2 = plsc.load_gather(gt_vm, [bvec + 2])
            gy2 = plsc.load_gather(gt_vm, [bvec + 3])
            ltx = jnp.maximum(gx1, px1)
            lty = jnp.maximum(gy1, py1)
            rbx = jnp.minimum(gx2, px2)
            rby = jnp.minimum(gy2, py2)
            wx = jnp.maximum(rbx - ltx, 0.0)
            wy = jnp.maximum(rby - lty, 0.0)
            overlap = wx * wy
            a1 = (gx2 - gx1) * (gy2 - gy1)
            a2 = (px2 - px1) * (py2 - py1)
            union = jnp.maximum(a1 + a2 - overlap, EPS)
            iou = overlap / union
            ex = jnp.maximum(jnp.maximum(gx2, px2) - jnp.minimum(gx1, px1), 0.0)
            ey = jnp.maximum(jnp.maximum(gy2, py2) - jnp.minimum(gy1, py1), 0.0)
            enclose = jnp.maximum(ex * ey, EPS)
            giou = iou - (enclose - union) / enclose
            gacc = gacc + (1.0 - giou)
            iou_v[j, sl] = iou

    for j in range(CH):
        pltpu.async_copy(iou_v.at[j], resp_hbm.at[sidx_v.at[j]], sem).wait()

    gacc_v[...] = gacc
    pltpu.sync_copy(gacc_v, gsum_hbm.at[wid])


_SC_MESH = plsc.VectorSubcoreMesh(
    core_axis_name="c", subcore_axis_name="s", num_cores=NC, num_subcores=NS)

_sc_call = pl.kernel(
    _sc_body,
    out_type=(
        jax.ShapeDtypeStruct((NC * TOT,), jnp.float32),
        jax.ShapeDtypeStruct((NW, L), jnp.float32),
    ),
    mesh=_SC_MESH,
    compiler_params=pltpu.CompilerParams(needs_layout_passes=False),
    scratch_types=[
        pltpu.VMEM((CH, CB), jnp.int32),
        pltpu.VMEM((CH, CB), jnp.int32),
        pltpu.VMEM((CH * 4, CB), jnp.int32),
        pltpu.VMEM((CH, CB), jnp.int32),
        pltpu.VMEM((CH * 4, CB), jnp.float32),
        pltpu.VMEM((CH, CB), jnp.float32),
        pltpu.VMEM((N * 4,), jnp.float32),
        pltpu.VMEM((L,), jnp.float32),
        pltpu.VMEM((ZROWS,), jnp.float32),
        pltpu.SemaphoreType.DMA,
    ],
)


def _tc_body(score_ref, resp_ref, gp_ref, np_ref, out_ref):
    s = score_ref[...]
    r = jnp.maximum(resp_ref[0], resp_ref[1])
    bce = jnp.maximum(s, 0.0) - s * r + jnp.log1p(jnp.exp(-jnp.abs(s)))
    tot = jnp.sum(bce) + jnp.sum(gp_ref[...])
    nps = jnp.maximum(np_ref[0], 1.0)
    out_ref[0] = tot / nps


def kernel(score_map, boxes, gt_boxes, num_positive_samples,
           positive_sample_batch_dim_indices, positive_sample_map_dim_indices):
    boxes_flat = boxes.reshape(TOT * 4).astype(jnp.float32)
    gt_flat = gt_boxes.reshape(N * 4).astype(jnp.float32)
    bidx = positive_sample_batch_dim_indices.astype(jnp.int32).reshape(NW * CH, CB)
    midx = positive_sample_map_dim_indices.astype(jnp.int32).reshape(NW * CH, CB)

    resp, gparts = _sc_call(boxes_flat, gt_flat, bidx, midx)

    score2 = score_map.reshape(TOT // 128, 128).astype(jnp.float32)
    resp3 = resp.reshape(NC, TOT // 128, 128)
    gp2 = gparts.reshape(4, 128)
    npos = num_positive_samples.astype(jnp.float32)

    out = pl.pallas_call(
        _tc_body,
        out_shape=jax.ShapeDtypeStruct((1,), jnp.float32),
        in_specs=[
            pl.BlockSpec(memory_space=pltpu.VMEM),
            pl.BlockSpec(memory_space=pltpu.VMEM),
            pl.BlockSpec(memory_space=pltpu.VMEM),
            pl.BlockSpec(memory_space=pltpu.SMEM),
        ],
        out_specs=pl.BlockSpec(memory_space=pltpu.SMEM),
    )(score2, resp3, gp2, npos)
    return out[0]


# speedup vs baseline: 1.0127x; 1.0127x over previous
import jax
import jax.numpy as jnp
from jax import lax
from jax.experimental import pallas as pl
from jax.experimental.pallas import tpu as pltpu
from jax.experimental.pallas import tpu_sc as plsc

N = 16
HW = 128 * 128
TOT = N * HW
P = 8192
NC, NS, L = 2, 16, 16
NW = NC * NS
PPW = P // NW
CH = 2
CB = PPW // CH
EPS = 1e-6
ZROWS = TOT // NS


def _sc_body(boxes_hbm, gt_hbm, bidx_hbm, midx_hbm, resp_hbm, gsum_hbm,
             bidx_v, midx_v, pidx_v, sidx_v, pc_v, iou_v, gt_vm,
             gacc_v, zbuf, sem, zsem):
    cid = lax.axis_index("c")
    sid = lax.axis_index("s")
    wid = sid * NC + cid

    cp_gt = pltpu.async_copy(gt_hbm, gt_vm, sem)
    cp_b = pltpu.async_copy(bidx_hbm.at[pl.ds(wid * CH, CH)], bidx_v, sem)
    cp_m = pltpu.async_copy(midx_hbm.at[pl.ds(wid * CH, CH)], midx_v, sem)

    def _z(i, c):
        for k in range(8):
            zbuf[pl.ds(i * 128 + k * L, L)] = jnp.zeros((L,), jnp.float32)
        return c
    lax.fori_loop(0, ZROWS // 128, _z, 0)
    cp_z = pltpu.async_copy(
        zbuf, resp_hbm.at[pl.ds(cid * TOT + sid * ZROWS, ZROWS)], zsem)

    cp_b.wait()
    cp_m.wait()

    for j in range(CH):
        for k in range(CB // L):
            b = bidx_v[j, pl.ds(k * L, L)]
            m = midx_v[j, pl.ds(k * L, L)]
            f = b * HW + m
            sidx_v[j, pl.ds(k * L, L)] = f + cid * TOT
            f4 = f * 4
            for c in range(4):
                pidx_v[j * 4 + c, pl.ds(k * L, L)] = f4 + c

    gathers = [
        pltpu.async_copy(boxes_hbm.at[pidx_v.at[j * 4 + c]],
                         pc_v.at[j * 4 + c], sem)
        for j in range(CH) for c in range(4)
    ]
    cp_gt.wait()
    for cp in gathers:
        cp.wait()

    gacc = jnp.zeros((L,), jnp.float32)
    for j in range(CH):
        for k in range(CB // L):
            sl = pl.ds(k * L, L)
            bvec = bidx_v[j, sl] * 4
            px1 = pc_v[j * 4 + 0, sl]
            py1 = pc_v[j * 4 + 1, sl]
            px2 = pc_v[j * 4 + 2, sl]
            py2 = pc_v[j * 4 + 3, sl]
            gx1 = plsc.load_gather(gt_vm, [bvec])
            gy1 = plsc.load_gather(gt_vm, [bvec + 1])
            gx2 = plsc.load_gather(gt_vm, [bvec + 2])
            gy2 = plsc.load_gather(gt_vm, [bvec + 3])
            ltx = jnp.maximum(gx1, px1)
            lty = jnp.maximum(gy1, py1)
            rbx = jnp.minimum(gx2, px2)
            rby = jnp.minimum(gy2, py2)
            wx = jnp.maximum(rbx - ltx, 0.0)
            wy = jnp.maximum(rby - lty, 0.0)
            overlap = wx * wy
            a1 = (gx2 - gx1) * (gy2 - gy1)
            a2 = (px2 - px1) * (py2 - py1)
            union = jnp.maximum(a1 + a2 - overlap, EPS)
            iou = overlap / union
            ex = jnp.maximum(jnp.maximum(gx2, px2) - jnp.minimum(gx1, px1), 0.0)
            ey = jnp.maximum(jnp.maximum(gy2, py2) - jnp.minimum(gy1, py1), 0.0)
            enclose = jnp.maximum(ex * ey, EPS)
            giou = iou - (enclose - union) / enclose
            gacc = gacc + (1.0 - giou)
            iou_v[j, sl] = iou

    gacc_v[...] = gacc
    cp_g = pltpu.async_copy(gacc_v, gsum_hbm.at[pl.ds(wid * L, L)], sem)

    cp_z.wait()
    plsc.subcore_barrier()

    scat = [
        pltpu.async_copy(iou_v.at[j], resp_hbm.at[sidx_v.at[j]], sem)
        for j in range(CH)
    ]
    cp_g.wait()
    for cp in scat:
        cp.wait()


_SC_MESH = plsc.VectorSubcoreMesh(
    core_axis_name="c", subcore_axis_name="s", num_cores=NC, num_subcores=NS)

_sc_call = pl.kernel(
    _sc_body,
    out_type=(
        jax.ShapeDtypeStruct((NC * TOT,), jnp.float32),
        jax.ShapeDtypeStruct((NW * L,), jnp.float32),
    ),
    mesh=_SC_MESH,
    compiler_params=pltpu.CompilerParams(needs_layout_passes=False),
    scratch_types=[
        pltpu.VMEM((CH, CB), jnp.int32),
        pltpu.VMEM((CH, CB), jnp.int32),
        pltpu.VMEM((CH * 4, CB), jnp.int32),
        pltpu.VMEM((CH, CB), jnp.int32),
        pltpu.VMEM((CH * 4, CB), jnp.float32),
        pltpu.VMEM((CH, CB), jnp.float32),
        pltpu.VMEM((N * 4,), jnp.float32),
        pltpu.VMEM((L,), jnp.float32),
        pltpu.VMEM((ZROWS,), jnp.float32),
        pltpu.SemaphoreType.DMA,
        pltpu.SemaphoreType.DMA,
    ],
)


def _tc_body(score_ref, resp_ref, gp_ref, np_ref, out_ref):
    s = score_ref[...]
    r = jnp.maximum(resp_ref[0], resp_ref[1])
    bce = jnp.maximum(s, 0.0) - s * r + jnp.log1p(jnp.exp(-jnp.abs(s)))
    tot = jnp.sum(bce) + jnp.sum(gp_ref[...])
    nps = jnp.maximum(np_ref[0], 1.0)
    out_ref[0] = tot / nps


def kernel(score_map, boxes, gt_boxes, num_positive_samples,
           positive_sample_batch_dim_indices, positive_sample_map_dim_indices):
    boxes_flat = boxes.reshape(TOT * 4).astype(jnp.float32)
    gt_flat = gt_boxes.reshape(N * 4).astype(jnp.float32)
    bidx = positive_sample_batch_dim_indices.astype(jnp.int32).reshape(NW * CH, CB)
    midx = positive_sample_map_dim_indices.astype(jnp.int32).reshape(NW * CH, CB)

    resp, gsum = _sc_call(boxes_flat, gt_flat, bidx, midx)

    score2 = score_map.reshape(TOT // 128, 128).astype(jnp.float32)
    resp3 = resp.reshape(NC, TOT // 128, 128)
    gp2 = gsum.reshape(4, 128)
    npos = num_positive_samples.astype(jnp.float32)

    out = pl.pallas_call(
        _tc_body,
        out_shape=jax.ShapeDtypeStruct((1,), jnp.float32),
        in_specs=[
            pl.BlockSpec(memory_space=pltpu.VMEM),
            pl.BlockSpec(memory_space=pltpu.VMEM),
            pl.BlockSpec(memory_space=pltpu.VMEM),
            pl.BlockSpec(memory_space=pltpu.SMEM),
        ],
        out_specs=pl.BlockSpec(memory_space=pltpu.SMEM),
    )(score2, resp3, gp2, npos)
    return out[0]


# speedup vs baseline: 5.5672x; 5.4974x over previous
import jax
import jax.numpy as jnp
from jax import lax
from jax.experimental import pallas as pl
from jax.experimental.pallas import tpu as pltpu
from jax.experimental.pallas import tpu_sc as plsc

N = 16
HW = 128 * 128
TOT = N * HW
P = 8192
NC, NS, L = 2, 16, 16
NW = NC * NS
PPW = P // NW
CH = 2
CB = PPW // CH
EPS = 1e-6
ZROWS = TOT // NS


def _sc_body(boxes_hbm, gt_hbm, bidx_hbm, midx_hbm, resp_hbm, gsum_hbm,
             bidx_v, midx_v, pidx_v, sidx_v, pc_v, iou_v, gt_vm,
             gacc_v, zbuf, sem, zsem):
    cid = lax.axis_index("c")
    sid = lax.axis_index("s")
    wid = sid * NC + cid

    cp_gt = pltpu.async_copy(gt_hbm, gt_vm, sem)
    cp_b = pltpu.async_copy(bidx_hbm.at[pl.ds(wid * CH, CH)], bidx_v, sem)
    cp_m = pltpu.async_copy(midx_hbm.at[pl.ds(wid * CH, CH)], midx_v, sem)

    def _z(i, c):
        for k in range(8):
            zbuf[pl.ds(i * 128 + k * L, L)] = jnp.zeros((L,), jnp.float32)
        return c
    lax.fori_loop(0, ZROWS // 128, _z, 0)
    cp_z = pltpu.async_copy(
        zbuf, resp_hbm.at[pl.ds(cid * TOT + sid * ZROWS, ZROWS)], zsem)

    cp_b.wait()
    cp_m.wait()

    for j in range(CH):
        for k in range(CB // L):
            b = bidx_v[j, pl.ds(k * L, L)]
            m = midx_v[j, pl.ds(k * L, L)]
            sidx_v[j, pl.ds(k * L, L)] = b * HW + m + cid * TOT
            base = b * 65536 + lax.shift_right_logical(m, 7) * 512 \
                + lax.bitwise_and(m, 127)
            for c in range(4):
                pidx_v[j * 4 + c, pl.ds(k * L, L)] = base + c * 128

    gathers = [
        pltpu.async_copy(boxes_hbm.at[pidx_v.at[j * 4 + c]],
                         pc_v.at[j * 4 + c], sem)
        for j in range(CH) for c in range(4)
    ]
    cp_gt.wait()
    for cp in gathers:
        cp.wait()

    gacc = jnp.zeros((L,), jnp.float32)
    for j in range(CH):
        for k in range(CB // L):
            sl = pl.ds(k * L, L)
            bvec = bidx_v[j, sl] * 4
            px1 = pc_v[j * 4 + 0, sl]
            py1 = pc_v[j * 4 + 1, sl]
            px2 = pc_v[j * 4 + 2, sl]
            py2 = pc_v[j * 4 + 3, sl]
            gx1 = plsc.load_gather(gt_vm, [bvec])
            gy1 = plsc.load_gather(gt_vm, [bvec + 1])
            gx2 = plsc.load_gather(gt_vm, [bvec + 2])
            gy2 = plsc.load_gather(gt_vm, [bvec + 3])
            ltx = jnp.maximum(gx1, px1)
            lty = jnp.maximum(gy1, py1)
            rbx = jnp.minimum(gx2, px2)
            rby = jnp.minimum(gy2, py2)
            wx = jnp.maximum(rbx - ltx, 0.0)
            wy = jnp.maximum(rby - lty, 0.0)
            overlap = wx * wy
            a1 = (gx2 - gx1) * (gy2 - gy1)
            a2 = (px2 - px1) * (py2 - py1)
            union = jnp.maximum(a1 + a2 - overlap, EPS)
            iou = overlap / union
            ex = jnp.maximum(jnp.maximum(gx2, px2) - jnp.minimum(gx1, px1), 0.0)
            ey = jnp.maximum(jnp.maximum(gy2, py2) - jnp.minimum(gy1, py1), 0.0)
            enclose = jnp.maximum(ex * ey, EPS)
            giou = iou - (enclose - union) / enclose
            gacc = gacc + (1.0 - giou)
            iou_v[j, sl] = iou

    gacc_v[...] = gacc
    cp_g = pltpu.async_copy(gacc_v, gsum_hbm.at[pl.ds(wid * L, L)], sem)

    cp_z.wait()
    plsc.subcore_barrier()

    scat = [
        pltpu.async_copy(iou_v.at[j], resp_hbm.at[sidx_v.at[j]], sem)
        for j in range(CH)
    ]
    cp_g.wait()
    for cp in scat:
        cp.wait()


_SC_MESH = plsc.VectorSubcoreMesh(
    core_axis_name="c", subcore_axis_name="s", num_cores=NC, num_subcores=NS)

_sc_call = pl.kernel(
    _sc_body,
    out_type=(
        jax.ShapeDtypeStruct((NC * TOT,), jnp.float32),
        jax.ShapeDtypeStruct((NW * L,), jnp.float32),
    ),
    mesh=_SC_MESH,
    compiler_params=pltpu.CompilerParams(needs_layout_passes=False),
    scratch_types=[
        pltpu.VMEM((CH, CB), jnp.int32),
        pltpu.VMEM((CH, CB), jnp.int32),
        pltpu.VMEM((CH * 4, CB), jnp.int32),
        pltpu.VMEM((CH, CB), jnp.int32),
        pltpu.VMEM((CH * 4, CB), jnp.float32),
        pltpu.VMEM((CH, CB), jnp.float32),
        pltpu.VMEM((N * 4,), jnp.float32),
        pltpu.VMEM((L,), jnp.float32),
        pltpu.VMEM((ZROWS,), jnp.float32),
        pltpu.SemaphoreType.DMA,
        pltpu.SemaphoreType.DMA,
    ],
)


def _tc_body(score_ref, resp_ref, gp_ref, np_ref, out_ref):
    s = score_ref[...]
    r = jnp.maximum(resp_ref[0], resp_ref[1])
    bce = jnp.maximum(s, 0.0) - s * r + jnp.log1p(jnp.exp(-jnp.abs(s)))
    tot = jnp.sum(bce) + jnp.sum(gp_ref[...])
    nps = jnp.maximum(np_ref[0], 1.0)
    out_ref[0] = tot / nps


def kernel(score_map, boxes, gt_boxes, num_positive_samples,
           positive_sample_batch_dim_indices, positive_sample_map_dim_indices):
    boxes_flat = (boxes.astype(jnp.float32)
                  .reshape(N, 128, 128, 4)
                  .transpose(0, 1, 3, 2)
                  .reshape(TOT * 4))
    gt_flat = gt_boxes.reshape(N * 4).astype(jnp.float32)
    bidx = positive_sample_batch_dim_indices.astype(jnp.int32).reshape(NW * CH, CB)
    midx = positive_sample_map_dim_indices.astype(jnp.int32).reshape(NW * CH, CB)

    resp, gsum = _sc_call(boxes_flat, gt_flat, bidx, midx)

    score2 = score_map.reshape(TOT // 128, 128).astype(jnp.float32)
    resp3 = resp.reshape(NC, TOT // 128, 128)
    gp2 = gsum.reshape(4, 128)
    npos = num_positive_samples.astype(jnp.float32)

    out = pl.pallas_call(
        _tc_body,
        out_shape=jax.ShapeDtypeStruct((1,), jnp.float32),
        in_specs=[
            pl.BlockSpec(memory_space=pltpu.VMEM),
            pl.BlockSpec(memory_space=pltpu.VMEM),
            pl.BlockSpec(memory_space=pltpu.VMEM),
            pl.BlockSpec(memory_space=pltpu.SMEM),
        ],
        out_specs=pl.BlockSpec(memory_space=pltpu.SMEM),
    )(score2, resp3, gp2, npos)
    return out[0]


# speedup vs baseline: 6.4260x; 1.1543x over previous
import jax
import jax.numpy as jnp
from jax import lax
from jax.experimental import pallas as pl
from jax.experimental.pallas import tpu as pltpu
from jax.experimental.pallas import tpu_sc as plsc

N = 16
HW = 128 * 128
TOT = N * HW
P = 8192
NC, NS, L = 2, 16, 16
NW = NC * NS
PPW = P // NW
CH = 2
CB = PPW // CH
PPC = NS * PPW
EPS = 1e-6
ZROWS = TOT // NS


def _sc_body(boxes_hbm, gt_hbm, bidx_hbm, midx_hbm, resp_hbm, gsum_hbm,
             pi_hbm, pf_hbm,
             bidx_v, midx_v, pidx_v, sidx_v, pc_v, iou_v, gt_vm, gacc_v,
             zbuf, sidx_all, iou_all, sem):
    cid = lax.axis_index("c")
    sid = lax.axis_index("s")
    wid = sid * NC + cid

    cp_gt = pltpu.async_copy(gt_hbm, gt_vm, sem)
    cp_b = pltpu.async_copy(bidx_hbm.at[pl.ds(wid * CH, CH)], bidx_v, sem)
    cp_m = pltpu.async_copy(midx_hbm.at[pl.ds(wid * CH, CH)], midx_v, sem)

    def _z(i, c):
        for k in range(8):
            zbuf[pl.ds(i * 128 + k * L, L)] = jnp.zeros((L,), jnp.float32)
        return c
    lax.fori_loop(0, ZROWS // 128, _z, 0)

    cp_b.wait()
    cp_m.wait()

    for j in range(CH):
        for k in range(CB // L):
            b = bidx_v[j, pl.ds(k * L, L)]
            m = midx_v[j, pl.ds(k * L, L)]
            sidx_v[pl.ds(j * CB + k * L, L)] = b * HW + m
            base = b * 65536 + lax.shift_right_logical(m, 7) * 512 \
                + lax.bitwise_and(m, 127)
            for c in range(4):
                pidx_v[j * 4 + c, pl.ds(k * L, L)] = base + c * 128

    gathers = [
        pltpu.async_copy(boxes_hbm.at[pidx_v.at[j * 4 + c]],
                         pc_v.at[j * 4 + c], sem)
        for j in range(CH) for c in range(4)
    ]
    cp_si = pltpu.async_copy(
        sidx_v, pi_hbm.at[pl.ds(cid * PPC + sid * PPW, PPW)], sem)
    for cp in gathers:
        cp.wait()
    cp_gt.wait()

    gacc = jnp.zeros((L,), jnp.float32)
    for j in range(CH):
        for k in range(CB // L):
            sl = pl.ds(k * L, L)
            bvec = bidx_v[j, sl] * 4
            px1 = pc_v[j * 4 + 0, sl]
            py1 = pc_v[j * 4 + 1, sl]
            px2 = pc_v[j * 4 + 2, sl]
            py2 = pc_v[j * 4 + 3, sl]
            gx1 = plsc.load_gather(gt_vm, [bvec])
            gy1 = plsc.load_gather(gt_vm, [bvec + 1])
            gx2 = plsc.load_gather(gt_vm, [bvec + 2])
            gy2 = plsc.load_gather(gt_vm, [bvec + 3])
            ltx = jnp.maximum(gx1, px1)
            lty = jnp.maximum(gy1, py1)
            rbx = jnp.minimum(gx2, px2)
            rby = jnp.minimum(gy2, py2)
            wx = jnp.maximum(rbx - ltx, 0.0)
            wy = jnp.maximum(rby - lty, 0.0)
            overlap = wx * wy
            a1 = (gx2 - gx1) * (gy2 - gy1)
            a2 = (px2 - px1) * (py2 - py1)
            union = jnp.maximum(a1 + a2 - overlap, EPS)
            iou = overlap / union
            ex = jnp.maximum(jnp.maximum(gx2, px2) - jnp.minimum(gx1, px1), 0.0)
            ey = jnp.maximum(jnp.maximum(gy2, py2) - jnp.minimum(gy1, py1), 0.0)
            enclose = jnp.maximum(ex * ey, EPS)
            giou = iou - (enclose - union) / enclose
            gacc = gacc + (1.0 - giou)
            iou_v[pl.ds(j * CB + k * L, L)] = iou

    gacc_v[...] = gacc
    cp_g = pltpu.async_copy(gacc_v, gsum_hbm.at[pl.ds(wid * L, L)], sem)
    cp_io = pltpu.async_copy(
        iou_v, pf_hbm.at[pl.ds(cid * PPC + sid * PPW, PPW)], sem)
    cp_si.wait()
    cp_io.wait()
    plsc.subcore_barrier()

    cp_rs = pltpu.async_copy(pi_hbm.at[pl.ds(cid * PPC, PPC)], sidx_all, sem)
    cp_ri = pltpu.async_copy(pf_hbm.at[pl.ds(cid * PPC, PPC)], iou_all, sem)
    cp_rs.wait()
    cp_ri.wait()

    lo = sid * ZROWS
    for f in range(NS):
        def _route(k, c, f=f):
            sl = pl.ds(f * PPW + k * L, L)
            sv = sidx_all[sl]
            iv = iou_all[sl]
            local = sv - lo
            mask = jnp.logical_and(sv >= lo, sv < lo + ZROWS)
            plsc.store_scatter(zbuf, [lax.bitwise_and(local, ZROWS - 1)],
                               iv, mask=mask)
            return c
        lax.fori_loop(0, PPW // L, _route, 0)

    pltpu.sync_copy(zbuf, resp_hbm.at[pl.ds(cid * TOT + sid * ZROWS, ZROWS)])
    cp_g.wait()


_SC_MESH = plsc.VectorSubcoreMesh(
    core_axis_name="c", subcore_axis_name="s", num_cores=NC, num_subcores=NS)

_sc_call = pl.kernel(
    _sc_body,
    out_type=(
        jax.ShapeDtypeStruct((NC * TOT,), jnp.float32),
        jax.ShapeDtypeStruct((NW * L,), jnp.float32),
        jax.ShapeDtypeStruct((NC * PPC,), jnp.int32),
        jax.ShapeDtypeStruct((NC * PPC,), jnp.float32),
    ),
    mesh=_SC_MESH,
    compiler_params=pltpu.CompilerParams(needs_layout_passes=False),
    scratch_types=[
        pltpu.VMEM((CH, CB), jnp.int32),
        pltpu.VMEM((CH, CB), jnp.int32),
        pltpu.VMEM((CH * 4, CB), jnp.int32),
        pltpu.VMEM((PPW,), jnp.int32),
        pltpu.VMEM((CH * 4, CB), jnp.float32),
        pltpu.VMEM((PPW,), jnp.float32),
        pltpu.VMEM((N * 4,), jnp.float32),
        pltpu.VMEM((L,), jnp.float32),
        pltpu.VMEM((ZROWS,), jnp.float32),
        pltpu.VMEM((PPC,), jnp.int32),
        pltpu.VMEM((PPC,), jnp.float32),
        pltpu.SemaphoreType.DMA,
    ],
)


def _tc_body(score_ref, resp_ref, gp_ref, np_ref, out_ref):
    s = score_ref[...]
    r = jnp.maximum(resp_ref[0], resp_ref[1])
    bce = jnp.maximum(s, 0.0) - s * r + jnp.log1p(jnp.exp(-jnp.abs(s)))
    tot = jnp.sum(bce) + jnp.sum(gp_ref[...])
    nps = jnp.maximum(np_ref[0], 1.0)
    out_ref[0] = tot / nps


def kernel(score_map, boxes, gt_boxes, num_positive_samples,
           positive_sample_batch_dim_indices, positive_sample_map_dim_indices):
    boxes_flat = (boxes.astype(jnp.float32)
                  .reshape(N, 128, 128, 4)
                  .transpose(0, 1, 3, 2)
                  .reshape(TOT * 4))
    gt_flat = gt_boxes.reshape(N * 4).astype(jnp.float32)
    bidx = positive_sample_batch_dim_indices.astype(jnp.int32).reshape(NW * CH, CB)
    midx = positive_sample_map_dim_indices.astype(jnp.int32).reshape(NW * CH, CB)

    resp, gsum, _, _ = _sc_call(boxes_flat, gt_flat, bidx, midx)

    score2 = score_map.reshape(TOT // 128, 128).astype(jnp.float32)
    resp3 = resp.reshape(NC, TOT // 128, 128)
    gp2 = gsum.reshape(4, 128)
    npos = num_positive_samples.astype(jnp.float32)

    out = pl.pallas_call(
        _tc_body,
        out_shape=jax.ShapeDtypeStruct((1,), jnp.float32),
        in_specs=[
            pl.BlockSpec(memory_space=pltpu.VMEM),
            pl.BlockSpec(memory_space=pltpu.VMEM),
            pl.BlockSpec(memory_space=pltpu.VMEM),
            pl.BlockSpec(memory_space=pltpu.SMEM),
        ],
        out_specs=pl.BlockSpec(memory_space=pltpu.SMEM),
    )(score2, resp3, gp2, npos)
    return out[0]
